# baseline (speedup 1.0000x reference)
import functools

import jax
import jax.numpy as jnp
from jax import lax
from jax.experimental import pallas as pl
from jax.experimental.pallas import tpu as pltpu
from jax.experimental.pallas import tpu_sc as plsc


def _cnorm_body(cb_ref, cnorm_ref):
    cb_size = cb_ref.shape[1]
    chunk = 1024
    for k in range(0, cb_size, chunk):
        blk = cb_ref[0, k:k + chunk, :]
        cnorm_ref[0, 0, k:k + chunk] = jnp.sum(blk * blk, axis=-1)


def _vq_body(cnorm_ref, x_ref, cb_ref, onehot_ref, idx_ref):
    ncb = cb_ref.shape[0]
    cb_size = cb_ref.shape[1]
    bn = x_ref.shape[0]
    chunk = 512
    for c in range(ncb):
        xc = x_ref[:, c, :]
        xnorm = jnp.sum(xc * xc, axis=-1, keepdims=True)
        bestv = None
        for k in range(0, cb_size, chunk):
            cbk = cb_ref[c, k:k + chunk, :]
            cnorm = cnorm_ref[c, 0, k:k + chunk][None, :]
            dotk = jnp.dot(xc, cbk.T, preferred_element_type=jnp.float32)
            distk = (xnorm + cnorm) - 2.0 * dotk
            mk = jnp.min(distk, axis=-1, keepdims=True)
            ak = jnp.argmin(distk, axis=-1)[:, None] + k
            if bestv is None:
                bestv, besti = mk, ak
            else:
                take = mk < bestv
                besti = jnp.where(take, ak, besti)
                bestv = jnp.where(take, mk, bestv)
        idx = besti[:, 0]
        iota = jax.lax.broadcasted_iota(jnp.int32, (bn, cb_size), 1)
        oh = (iota == idx[:, None]).astype(jnp.float32)
        onehot_ref[:, c, :] = oh
        idx_ref[:, c, :] = idx[:, None]


def _make_sc_gather(rows, ncb, cb_size, dim, b_per_w):
    mesh = plsc.VectorSubcoreMesh(core_axis_name="c", subcore_axis_name="s")
    nc = plsc.get_sparse_core_info().num_cores

    @functools.partial(
        pl.kernel,
        mesh=mesh,
        compiler_params=pltpu.CompilerParams(use_tc_tiling_on_sc=False),
        out_type=jax.ShapeDtypeStruct((rows, dim), jnp.float32),
        scratch_types=[
            pltpu.VMEM((b_per_w,), jnp.int32),
            pltpu.VMEM((b_per_w,), jnp.int32),
            pltpu.VMEM((b_per_w, dim), jnp.float32),
            pltpu.SemaphoreType.DMA,
        ],
    )
    def sc_gather(table_hbm, idx_hbm, out_hbm, idx_v, flat_v, rows_v, sem):
        wid = lax.axis_index("s") * nc + lax.axis_index("c")
        base = wid * b_per_w
        pltpu.sync_copy(idx_hbm.at[pl.ds(base, b_per_w)], idx_v)
        lane = lax.iota(jnp.int32, 16)

        def body(j, carry):
            v = idx_v[pl.ds(j * 16, 16)]
            r0 = base + j * 16
            off = ((r0 + lane) % ncb) * cb_size
            flat_v[pl.ds(j * 16, 16)] = v + off
            return carry

        lax.fori_loop(0, b_per_w // 16, body, 0)
        pltpu.async_copy(table_hbm.at[flat_v], rows_v, sem).wait()
        pltpu.sync_copy(rows_v, out_hbm.at[pl.ds(base, b_per_w)])

    return sc_gather


@functools.partial(jax.jit, static_argnames=("block_n",))
def _vq(x, codebook, block_n=128):
    n, ncb, dim = x.shape
    _, cb_size, _ = codebook.shape

    cnorm = pl.pallas_call(
        _cnorm_body,
        grid=(ncb,),
        in_specs=[pl.BlockSpec((1, cb_size, dim), lambda c: (c, 0, 0))],
        out_specs=pl.BlockSpec((1, 1, cb_size), lambda c: (c, 0, 0)),
        out_shape=jax.ShapeDtypeStruct((ncb, 1, cb_size), jnp.float32),
    )(codebook)

    grid = (n // block_n,)
    out_shapes = (
        jax.ShapeDtypeStruct((n, ncb, cb_size), jnp.float32),
        jax.ShapeDtypeStruct((n, ncb, 1), jnp.int32),
    )
    out_specs = (
        pl.BlockSpec((block_n, ncb, cb_size), lambda i: (i, 0, 0)),
        pl.BlockSpec((block_n, ncb, 1), lambda i: (i, 0, 0)),
    )
    in_specs = [
        pl.BlockSpec((ncb, 1, cb_size), lambda i: (0, 0, 0)),
        pl.BlockSpec((block_n, ncb, dim), lambda i: (i, 0, 0)),
        pl.BlockSpec((ncb, cb_size, dim), lambda i: (0, 0, 0)),
    ]
    one_hot, index = pl.pallas_call(
        _vq_body,
        grid=grid,
        in_specs=in_specs,
        out_specs=out_specs,
        out_shape=out_shapes,
    )(cnorm, x, codebook)

    rows = n * ncb
    nw = 32
    b_per_w = rows // nw
    table = codebook.reshape(ncb * cb_size, dim)
    idx_flat = index.reshape(rows)
    x_hat = _make_sc_gather(rows, ncb, cb_size, dim, b_per_w)(table, idx_flat)
    x_hat = x_hat.reshape(n, ncb, dim)
    return (x_hat, one_hot, index)


def kernel(x, codebook):
    return _vq(x, codebook)


# speedup vs baseline: 3.2929x; 3.2929x over previous
import functools

import jax
import jax.numpy as jnp
from jax import lax
from jax.experimental import pallas as pl
from jax.experimental.pallas import tpu as pltpu
from jax.experimental.pallas import tpu_sc as plsc


def _cnorm_body(cb_ref, cnorm_ref):
    cb_size = cb_ref.shape[1]
    chunk = 1024
    for k in range(0, cb_size, chunk):
        blk = cb_ref[0, k:k + chunk, :]
        cnorm_ref[0, 0, k:k + chunk] = jnp.sum(blk * blk, axis=-1)


def _argmin_body(cnorm_ref, x_ref, cb_ref, idx_ref):
    ncb = cb_ref.shape[0]
    for c in range(ncb):
        cbc = cb_ref[c]
        xc = x_ref[:, c, :]
        cnorm = cnorm_ref[c, 0, :][None, :]
        xnorm = jnp.sum(xc * xc, axis=-1, keepdims=True)
        dot2 = jnp.dot(-2.0 * xc, cbc.T, preferred_element_type=jnp.float32)
        dist = (xnorm + cnorm) + dot2
        idx = jnp.argmin(dist, axis=-1)
        idx_ref[:, c, :] = idx[:, None]


def _onehot_body(idx_ref, onehot_ref):
    ncb = idx_ref.shape[1]
    bn = idx_ref.shape[0]
    cb_size = onehot_ref.shape[2]
    iota = jax.lax.broadcasted_iota(jnp.int32, (bn, cb_size), 1)
    for c in range(ncb):
        idx = idx_ref[:, c, :]
        onehot_ref[:, c, :] = (iota == idx).astype(jnp.float32)


def _make_sc_gather(rows, ncb, cb_size, dim, b_per_w):
    mesh = plsc.VectorSubcoreMesh(core_axis_name="c", subcore_axis_name="s")
    nc = plsc.get_sparse_core_info().num_cores

    @functools.partial(
        pl.kernel,
        mesh=mesh,
        compiler_params=pltpu.CompilerParams(use_tc_tiling_on_sc=False),
        out_type=jax.ShapeDtypeStruct((rows, dim), jnp.float32),
        scratch_types=[
            pltpu.VMEM((b_per_w,), jnp.int32),
            pltpu.VMEM((b_per_w,), jnp.int32),
            pltpu.VMEM((b_per_w, dim), jnp.float32),
            pltpu.SemaphoreType.DMA,
        ],
    )
    def sc_gather(table_hbm, idx_hbm, out_hbm, idx_v, flat_v, rows_v, sem):
        wid = lax.axis_index("s") * nc + lax.axis_index("c")
        base = wid * b_per_w
        pltpu.sync_copy(idx_hbm.at[pl.ds(base, b_per_w)], idx_v)
        lane = lax.iota(jnp.int32, 16)

        def body(j, carry):
            v = idx_v[pl.ds(j * 16, 16)]
            r0 = base + j * 16
            off = ((r0 + lane) % ncb) * cb_size
            flat_v[pl.ds(j * 16, 16)] = v + off
            return carry

        lax.fori_loop(0, b_per_w // 16, body, 0)
        pltpu.async_copy(table_hbm.at[flat_v], rows_v, sem).wait()
        pltpu.sync_copy(rows_v, out_hbm.at[pl.ds(base, b_per_w)])

    return sc_gather


@functools.partial(jax.jit, static_argnames=("block_n", "block_oh"))
def _vq(x, codebook, block_n=128, block_oh=128):
    n, ncb, dim = x.shape
    _, cb_size, _ = codebook.shape

    cnorm = pl.pallas_call(
        _cnorm_body,
        grid=(ncb,),
        in_specs=[pl.BlockSpec((1, cb_size, dim), lambda c: (c, 0, 0))],
        out_specs=pl.BlockSpec((1, 1, cb_size), lambda c: (c, 0, 0)),
        out_shape=jax.ShapeDtypeStruct((ncb, 1, cb_size), jnp.float32),
    )(codebook)

    index = pl.pallas_call(
        _argmin_body,
        grid=(n // block_n,),
        in_specs=[
            pl.BlockSpec((ncb, 1, cb_size), lambda i: (0, 0, 0)),
            pl.BlockSpec((block_n, ncb, dim), lambda i: (i, 0, 0)),
            pl.BlockSpec((ncb, cb_size, dim), lambda i: (0, 0, 0)),
        ],
        out_specs=pl.BlockSpec((block_n, ncb, 1), lambda i: (i, 0, 0)),
        out_shape=jax.ShapeDtypeStruct((n, ncb, 1), jnp.int32),
    )(cnorm, x, codebook)

    one_hot = pl.pallas_call(
        _onehot_body,
        grid=(n // block_oh,),
        in_specs=[pl.BlockSpec((block_oh, ncb, 1), lambda i: (i, 0, 0))],
        out_specs=pl.BlockSpec((block_oh, ncb, cb_size), lambda i: (i, 0, 0)),
        out_shape=jax.ShapeDtypeStruct((n, ncb, cb_size), jnp.float32),
    )(index)

    rows = n * ncb
    nw = 32
    b_per_w = rows // nw
    table = codebook.reshape(ncb * cb_size, dim)
    idx_flat = index.reshape(rows)
    x_hat = _make_sc_gather(rows, ncb, cb_size, dim, b_per_w)(table, idx_flat)
    x_hat = x_hat.reshape(n, ncb, dim)
    return (x_hat, one_hot, index)


def kernel(x, codebook):
    return _vq(x, codebook)


# speedup vs baseline: 3.3481x; 1.0168x over previous
import functools

import jax
import jax.numpy as jnp
from jax import lax
from jax.experimental import pallas as pl
from jax.experimental.pallas import tpu as pltpu
from jax.experimental.pallas import tpu_sc as plsc


def _cnorm_body(cb_ref, cnorm_ref):
    cb_size = cb_ref.shape[1]
    chunk = 1024
    for k in range(0, cb_size, chunk):
        blk = cb_ref[0, k:k + chunk, :]
        cnorm_ref[0, 0, k:k + chunk] = jnp.sum(blk * blk, axis=-1)


def _argmin_body(cnorm_ref, x_ref, cb_ref, idx_ref):
    ncb = cb_ref.shape[0]
    for c in range(ncb):
        cbc = cb_ref[c]
        xc = x_ref[:, c, :]
        cnorm = cnorm_ref[c, 0, :][None, :]
        xnorm = jnp.sum(xc * xc, axis=-1, keepdims=True)
        dot2 = jnp.dot(-2.0 * xc, cbc.T, preferred_element_type=jnp.float32)
        dist = (xnorm + cnorm) + dot2
        idx = jnp.argmin(dist, axis=-1)
        idx_ref[:, c, :] = idx[:, None]


def _onehot_body(idx_ref, onehot_ref):
    ncb = idx_ref.shape[1]
    bn = idx_ref.shape[0]
    cb_size = onehot_ref.shape[2]
    iota = jax.lax.broadcasted_iota(jnp.int32, (bn, cb_size), 1)
    for c in range(ncb):
        idx = idx_ref[:, c, :]
        onehot_ref[:, c, :] = (iota == idx).astype(jnp.float32)


def _make_sc_gather(rows, ncb, cb_size, dim, b_per_w):
    mesh = plsc.VectorSubcoreMesh(core_axis_name="c", subcore_axis_name="s")
    nc = plsc.get_sparse_core_info().num_cores

    @functools.partial(
        pl.kernel,
        mesh=mesh,
        compiler_params=pltpu.CompilerParams(use_tc_tiling_on_sc=False),
        out_type=jax.ShapeDtypeStruct((rows, dim), jnp.float32),
        scratch_types=[
            pltpu.VMEM((b_per_w,), jnp.int32),
            pltpu.VMEM((b_per_w,), jnp.int32),
            pltpu.VMEM((b_per_w, dim), jnp.float32),
            pltpu.SemaphoreType.DMA,
        ],
    )
    def sc_gather(table_hbm, idx_hbm, out_hbm, idx_v, flat_v, rows_v, sem):
        wid = lax.axis_index("s") * nc + lax.axis_index("c")
        base = wid * b_per_w
        pltpu.sync_copy(idx_hbm.at[pl.ds(base, b_per_w)], idx_v)
        lane = lax.iota(jnp.int32, 16)

        def body(j, carry):
            v = idx_v[pl.ds(j * 16, 16)]
            r0 = base + j * 16
            off = ((r0 + lane) % ncb) * cb_size
            flat_v[pl.ds(j * 16, 16)] = v + off
            return carry

        lax.fori_loop(0, b_per_w // 16, body, 0)
        pltpu.async_copy(table_hbm.at[flat_v], rows_v, sem).wait()
        pltpu.sync_copy(rows_v, out_hbm.at[pl.ds(base, b_per_w)])

    return sc_gather


@functools.partial(jax.jit, static_argnames=("block_n", "block_oh"))
def _vq(x, codebook, block_n=256, block_oh=128):
    n, ncb, dim = x.shape
    _, cb_size, _ = codebook.shape

    cnorm = pl.pallas_call(
        _cnorm_body,
        grid=(ncb,),
        in_specs=[pl.BlockSpec((1, cb_size, dim), lambda c: (c, 0, 0))],
        out_specs=pl.BlockSpec((1, 1, cb_size), lambda c: (c, 0, 0)),
        out_shape=jax.ShapeDtypeStruct((ncb, 1, cb_size), jnp.float32),
    )(codebook)

    index = pl.pallas_call(
        _argmin_body,
        grid=(n // block_n,),
        in_specs=[
            pl.BlockSpec((ncb, 1, cb_size), lambda i: (0, 0, 0)),
            pl.BlockSpec((block_n, ncb, dim), lambda i: (i, 0, 0)),
            pl.BlockSpec((ncb, cb_size, dim), lambda i: (0, 0, 0)),
        ],
        out_specs=pl.BlockSpec((block_n, ncb, 1), lambda i: (i, 0, 0)),
        out_shape=jax.ShapeDtypeStruct((n, ncb, 1), jnp.int32),
    )(cnorm, x, codebook)

    rows = n * ncb
    nw = 32
    b_per_w = rows // nw
    table = codebook.reshape(ncb * cb_size, dim)
    idx_flat = index.reshape(rows)
    x_hat = _make_sc_gather(rows, ncb, cb_size, dim, b_per_w)(table, idx_flat)
    x_hat = x_hat.reshape(n, ncb, dim)

    one_hot = pl.pallas_call(
        _onehot_body,
        grid=(n // block_oh,),
        in_specs=[pl.BlockSpec((block_oh, ncb, 1), lambda i: (i, 0, 0))],
        out_specs=pl.BlockSpec((block_oh, ncb, cb_size), lambda i: (i, 0, 0)),
        out_shape=jax.ShapeDtypeStruct((n, ncb, cb_size), jnp.float32),
    )(index)
    return (x_hat, one_hot, index)


def kernel(x, codebook):
    return _vq(x, codebook)


# speedup vs baseline: 4.0643x; 1.2139x over previous
import functools

import jax
import jax.numpy as jnp
from jax import lax
from jax.experimental import pallas as pl
from jax.experimental.pallas import tpu as pltpu
from jax.experimental.pallas import tpu_sc as plsc


def _cnorm_body(cb_ref, cnorm_ref):
    cb_size = cb_ref.shape[1]
    chunk = 1024
    for k in range(0, cb_size, chunk):
        blk = cb_ref[0, k:k + chunk, :]
        cnorm_ref[0, 0, k:k + chunk] = jnp.sum(blk * blk, axis=-1)


def _argmin_body(cnorm_ref, x_ref, cb_ref, idx_ref):
    ncb = cb_ref.shape[0]
    for c in range(ncb):
        cbc = cb_ref[c]
        xc = x_ref[:, c, :]
        cnorm = cnorm_ref[c, 0, :][None, :]
        xnorm = jnp.sum(xc * xc, axis=-1, keepdims=True)
        dot2 = jnp.dot(-2.0 * xc, cbc.T, preferred_element_type=jnp.float32)
        dist = (xnorm + cnorm) + dot2
        idx = jnp.argmin(dist, axis=-1)
        idx_ref[:, c, :] = idx[:, None]


def _onehot_body(idx_ref, cb_ref, onehot_ref, xhat_ref):
    ncb = idx_ref.shape[1]
    bn = idx_ref.shape[0]
    cb_size = onehot_ref.shape[2]
    iota = jax.lax.broadcasted_iota(jnp.int32, (bn, cb_size), 1)
    for c in range(ncb):
        idx = idx_ref[:, c, :]
        oh = (iota == idx).astype(jnp.float32)
        onehot_ref[:, c, :] = oh
        xhat_ref[:, c, :] = jnp.dot(oh, cb_ref[c],
                                    preferred_element_type=jnp.float32)


def _make_sc_gather(rows, ncb, cb_size, dim, b_per_w):
    mesh = plsc.VectorSubcoreMesh(core_axis_name="c", subcore_axis_name="s")
    nc = plsc.get_sparse_core_info().num_cores

    @functools.partial(
        pl.kernel,
        mesh=mesh,
        compiler_params=pltpu.CompilerParams(use_tc_tiling_on_sc=False),
        out_type=jax.ShapeDtypeStruct((rows, dim), jnp.float32),
        scratch_types=[
            pltpu.VMEM((b_per_w,), jnp.int32),
            pltpu.VMEM((b_per_w,), jnp.int32),
            pltpu.VMEM((b_per_w, dim), jnp.float32),
            pltpu.SemaphoreType.DMA,
        ],
    )
    def sc_gather(table_hbm, idx_hbm, out_hbm, idx_v, flat_v, rows_v, sem):
        wid = lax.axis_index("s") * nc + lax.axis_index("c")
        base = wid * b_per_w
        pltpu.sync_copy(idx_hbm.at[pl.ds(base, b_per_w)], idx_v)
        lane = lax.iota(jnp.int32, 16)

        def body(j, carry):
            v = idx_v[pl.ds(j * 16, 16)]
            r0 = base + j * 16
            off = ((r0 + lane) % ncb) * cb_size
            flat_v[pl.ds(j * 16, 16)] = v + off
            return carry

        lax.fori_loop(0, b_per_w // 16, body, 0)
        pltpu.async_copy(table_hbm.at[flat_v], rows_v, sem).wait()
        pltpu.sync_copy(rows_v, out_hbm.at[pl.ds(base, b_per_w)])

    return sc_gather


@functools.partial(jax.jit, static_argnames=("block_n", "block_oh"))
def _vq(x, codebook, block_n=256, block_oh=128):
    n, ncb, dim = x.shape
    _, cb_size, _ = codebook.shape

    cnorm = pl.pallas_call(
        _cnorm_body,
        grid=(ncb,),
        in_specs=[pl.BlockSpec((1, cb_size, dim), lambda c: (c, 0, 0))],
        out_specs=pl.BlockSpec((1, 1, cb_size), lambda c: (c, 0, 0)),
        out_shape=jax.ShapeDtypeStruct((ncb, 1, cb_size), jnp.float32),
    )(codebook)

    index = pl.pallas_call(
        _argmin_body,
        grid=(n // block_n,),
        in_specs=[
            pl.BlockSpec((ncb, 1, cb_size), lambda i: (0, 0, 0)),
            pl.BlockSpec((block_n, ncb, dim), lambda i: (i, 0, 0)),
            pl.BlockSpec((ncb, cb_size, dim), lambda i: (0, 0, 0)),
        ],
        out_specs=pl.BlockSpec((block_n, ncb, 1), lambda i: (i, 0, 0)),
        out_shape=jax.ShapeDtypeStruct((n, ncb, 1), jnp.int32),
    )(cnorm, x, codebook)

    one_hot, x_hat = pl.pallas_call(
        _onehot_body,
        grid=(n // block_oh,),
        in_specs=[
            pl.BlockSpec((block_oh, ncb, 1), lambda i: (i, 0, 0)),
            pl.BlockSpec((ncb, cb_size, dim), lambda i: (0, 0, 0)),
        ],
        out_specs=(
            pl.BlockSpec((block_oh, ncb, cb_size), lambda i: (i, 0, 0)),
            pl.BlockSpec((block_oh, ncb, dim), lambda i: (i, 0, 0)),
        ),
        out_shape=(
            jax.ShapeDtypeStruct((n, ncb, cb_size), jnp.float32),
            jax.ShapeDtypeStruct((n, ncb, dim), jnp.float32),
        ),
    )(index, codebook)
    return (x_hat, one_hot, index)


def kernel(x, codebook):
    return _vq(x, codebook)


# speedup vs baseline: 4.1255x; 1.0151x over previous
import functools

import jax
import jax.numpy as jnp
from jax import lax
from jax.experimental import pallas as pl
from jax.experimental.pallas import tpu as pltpu
from jax.experimental.pallas import tpu_sc as plsc


def _cnorm_body(cb_ref, cnorm_ref):
    cb_size = cb_ref.shape[1]
    chunk = 1024
    for k in range(0, cb_size, chunk):
        blk = cb_ref[0, k:k + chunk, :]
        cnorm_ref[0, 0, k:k + chunk] = jnp.sum(blk * blk, axis=-1)


def _argmin_body(cnorm_ref, x_ref, cb_ref, idx_ref):
    ncb = cb_ref.shape[0]
    cb_size = cb_ref.shape[1]

    @pl.when(pl.program_id(0) == 0)
    def _():
        chunk = 1024
        for c in range(ncb):
            for k in range(0, cb_size, chunk):
                blk = cb_ref[c, k:k + chunk, :]
                cnorm_ref[c, 0, k:k + chunk] = jnp.sum(blk * blk, axis=-1)

    for c in range(ncb):
        cbc = cb_ref[c]
        xc = x_ref[:, c, :]
        cnorm = cnorm_ref[c, 0, :][None, :]
        xnorm = jnp.sum(xc * xc, axis=-1, keepdims=True)
        dot2 = jnp.dot(-2.0 * xc, cbc.T, preferred_element_type=jnp.float32)
        dist = (xnorm + cnorm) + dot2
        idx = jnp.argmin(dist, axis=-1)
        idx_ref[:, c, :] = idx[:, None]


def _onehot_body(idx_ref, cb_ref, onehot_ref, xhat_ref):
    ncb = idx_ref.shape[1]
    bn = idx_ref.shape[0]
    cb_size = onehot_ref.shape[2]
    iota = jax.lax.broadcasted_iota(jnp.int32, (bn, cb_size), 1)
    for c in range(ncb):
        idx = idx_ref[:, c, :]
        oh = (iota == idx).astype(jnp.float32)
        onehot_ref[:, c, :] = oh
        xhat_ref[:, c, :] = jnp.dot(oh, cb_ref[c],
                                    preferred_element_type=jnp.float32)


def _make_sc_gather(rows, ncb, cb_size, dim, b_per_w):
    mesh = plsc.VectorSubcoreMesh(core_axis_name="c", subcore_axis_name="s")
    nc = plsc.get_sparse_core_info().num_cores

    @functools.partial(
        pl.kernel,
        mesh=mesh,
        compiler_params=pltpu.CompilerParams(use_tc_tiling_on_sc=False),
        out_type=jax.ShapeDtypeStruct((rows, dim), jnp.float32),
        scratch_types=[
            pltpu.VMEM((b_per_w,), jnp.int32),
            pltpu.VMEM((b_per_w,), jnp.int32),
            pltpu.VMEM((b_per_w, dim), jnp.float32),
            pltpu.SemaphoreType.DMA,
        ],
    )
    def sc_gather(table_hbm, idx_hbm, out_hbm, idx_v, flat_v, rows_v, sem):
        wid = lax.axis_index("s") * nc + lax.axis_index("c")
        base = wid * b_per_w
        pltpu.sync_copy(idx_hbm.at[pl.ds(base, b_per_w)], idx_v)
        lane = lax.iota(jnp.int32, 16)

        def body(j, carry):
            v = idx_v[pl.ds(j * 16, 16)]
            r0 = base + j * 16
            off = ((r0 + lane) % ncb) * cb_size
            flat_v[pl.ds(j * 16, 16)] = v + off
            return carry

        lax.fori_loop(0, b_per_w // 16, body, 0)
        pltpu.async_copy(table_hbm.at[flat_v], rows_v, sem).wait()
        pltpu.sync_copy(rows_v, out_hbm.at[pl.ds(base, b_per_w)])

    return sc_gather


@functools.partial(jax.jit, static_argnames=("block_n", "block_oh"))
def _vq(x, codebook, block_n=256, block_oh=128):
    n, ncb, dim = x.shape
    _, cb_size, _ = codebook.shape

    index = pl.pallas_call(
        lambda x_ref, cb_ref, idx_ref, cnorm_ref: _argmin_body(
            cnorm_ref, x_ref, cb_ref, idx_ref),
        grid=(n // block_n,),
        in_specs=[
            pl.BlockSpec((block_n, ncb, dim), lambda i: (i, 0, 0)),
            pl.BlockSpec((ncb, cb_size, dim), lambda i: (0, 0, 0)),
        ],
        out_specs=pl.BlockSpec((block_n, ncb, 1), lambda i: (i, 0, 0)),
        out_shape=jax.ShapeDtypeStruct((n, ncb, 1), jnp.int32),
        scratch_shapes=[pltpu.VMEM((ncb, 1, cb_size), jnp.float32)],
    )(x, codebook)

    one_hot, x_hat = pl.pallas_call(
        _onehot_body,
        grid=(n // block_oh,),
        in_specs=[
            pl.BlockSpec((block_oh, ncb, 1), lambda i: (i, 0, 0)),
            pl.BlockSpec((ncb, cb_size, dim), lambda i: (0, 0, 0)),
        ],
        out_specs=(
            pl.BlockSpec((block_oh, ncb, cb_size), lambda i: (i, 0, 0)),
            pl.BlockSpec((block_oh, ncb, dim), lambda i: (i, 0, 0)),
        ),
        out_shape=(
            jax.ShapeDtypeStruct((n, ncb, cb_size), jnp.float32),
            jax.ShapeDtypeStruct((n, ncb, dim), jnp.float32),
        ),
    )(index, codebook)
    return (x_hat, one_hot, index)


def kernel(x, codebook):
    return _vq(x, codebook)
